# baseline (speedup 1.0000x reference)
import functools

import jax
import jax.numpy as jnp
from jax import lax
from jax.experimental import pallas as pl
from jax.experimental.pallas import tpu as pltpu
from jax.experimental.pallas import tpu_sc as plsc

E = 3_200_000
EMB = 16
OUT_D = 32
NC = 2
NS = 16
L = 16
NW = NC * NS
EB = E // 128
BPC = 25
CHUNK_E = BPC * 128
IW = BPC * 256
OW = BPC * 1024
NCHUNKS = EB // BPC
CPW = -(-NCHUNKS // NW)
GROUPS = CHUNK_E // L

_mesh = plsc.VectorSubcoreMesh(core_axis_name="c", subcore_axis_name="s")


@functools.partial(
    pl.kernel,
    mesh=_mesh,
    compiler_params=pltpu.CompilerParams(
        needs_layout_passes=False, use_tc_tiling_on_sc=False
    ),
    out_type=jax.ShapeDtypeStruct((E * OUT_D,), jnp.float32),
    scratch_types=[
        pltpu.VMEM((4, EMB), jnp.float32),
        pltpu.VMEM((4, EMB), jnp.float32),
        pltpu.VMEM((IW,), jnp.int32),
        pltpu.VMEM((4 * OW,), jnp.float32),
    ],
)
def _edge_encode(edge_hbm, w0_hbm, w1_hbm, out_hbm, w0_v, w1_v, idx_v, out_v):
    wid = lax.axis_index("s") * NC + lax.axis_index("c")
    pltpu.sync_copy(w0_hbm, w0_v)
    pltpu.sync_copy(w1_hbm, w1_v)

    iota = lax.iota(jnp.int32, L)

    def chunk_body(k, carry):
        ci = k * NW + wid

        @pl.when(ci < NCHUNKS)
        def _():
            t0 = ci * BPC
            pltpu.sync_copy(edge_hbm.at[pl.ds(ci * IW, IW)], idx_v)

            @plsc.parallel_loop(0, GROUPS, 1, unroll=2)
            def _grp(g):
                b = g >> 3
                ll0 = (g & 7) << 4
                in_addr = (b * 256 + ll0) + iota
                a0 = plsc.load_gather(idx_v, [in_addr])
                a1 = plsc.load_gather(idx_v, [in_addr + 128])
                ob = (b * 1024 + ll0) + iota
                for d in range(OUT_D):
                    r, m = d >> 3, d & 7
                    cc = jnp.full((L,), d % EMB, jnp.int32)
                    if d < EMB:
                        v = plsc.load_gather(w0_v, [a0, cc])
                    else:
                        v = plsc.load_gather(w1_v, [a1, cc])
                    plsc.store_scatter(out_v, [ob + (r * OW + m * 128)], v)

            for r in range(4):
                pltpu.sync_copy(
                    out_v.at[pl.ds(r * OW, OW)],
                    out_hbm.at[pl.ds((r * EB + t0) * 1024, OW)],
                )

        return carry

    lax.fori_loop(0, CPW, chunk_body, 0)


def kernel(edge_attr, W0, W1):
    ea_lin = edge_attr.reshape(EB, 128, 2).transpose(0, 2, 1).reshape(E * 2)
    flat = _edge_encode(ea_lin, W0, W1)
    return flat.reshape(4, EB, 8, 128).transpose(1, 3, 0, 2).reshape(E, OUT_D)


# speedup vs baseline: 1.3108x; 1.3108x over previous
import functools

import jax
import jax.numpy as jnp
from jax import lax
from jax.experimental import pallas as pl
from jax.experimental.pallas import tpu as pltpu
from jax.experimental.pallas import tpu_sc as plsc

E = 3_200_000
EMB = 16
OUT_D = 32
NC = 2
NS = 16
L = 16
NW = NC * NS
EB = E // 128
BPC = 8
CHUNK_E = BPC * 128
IW = BPC * 256
OW = BPC * 1024
NCHUNKS = EB // BPC
CPW = -(-NCHUNKS // NW)
GROUPS = CHUNK_E // L

_mesh = plsc.VectorSubcoreMesh(core_axis_name="c", subcore_axis_name="s")


@functools.partial(
    pl.kernel,
    mesh=_mesh,
    compiler_params=pltpu.CompilerParams(
        needs_layout_passes=False, use_tc_tiling_on_sc=False
    ),
    out_type=jax.ShapeDtypeStruct((E * OUT_D,), jnp.float32),
    scratch_types=[
        pltpu.VMEM((4, EMB), jnp.float32),
        pltpu.VMEM((4, EMB), jnp.float32),
        pltpu.VMEM((IW,), jnp.int32),
        pltpu.VMEM((4 * OW,), jnp.float32),
    ],
)
def _edge_encode(edge_hbm, w0_hbm, w1_hbm, out_hbm, w0_v, w1_v, idx_v, out_v):
    wid = lax.axis_index("s") * NC + lax.axis_index("c")
    pltpu.sync_copy(w0_hbm, w0_v)
    pltpu.sync_copy(w1_hbm, w1_v)

    iota = lax.iota(jnp.int32, L)

    def chunk_body(k, carry):
        ci = k * NW + wid

        @pl.when(ci < NCHUNKS)
        def _():
            t0 = ci * BPC
            pltpu.sync_copy(edge_hbm.at[pl.ds(ci * IW, IW)], idx_v)

            @plsc.parallel_loop(0, GROUPS, 1, unroll=2)
            def _grp(g):
                b = g >> 3
                ll0 = (g & 7) << 4
                ib = b * 256 + ll0
                a0 = idx_v[pl.ds(ib, L)]
                a1 = idx_v[pl.ds(ib + 128, L)]
                ob = b * 1024 + ll0
                for d in range(OUT_D):
                    r, m = d >> 3, d & 7
                    cc = jnp.full((L,), d % EMB, jnp.int32)
                    if d < EMB:
                        v = plsc.load_gather(w0_v, [a0, cc])
                    else:
                        v = plsc.load_gather(w1_v, [a1, cc])
                    out_v[pl.ds(ob + (r * OW + m * 128), L)] = v

            for r in range(4):
                pltpu.sync_copy(
                    out_v.at[pl.ds(r * OW, OW)],
                    out_hbm.at[pl.ds((r * EB + t0) * 1024, OW)],
                )

        return carry

    lax.fori_loop(0, CPW, chunk_body, 0)


def kernel(edge_attr, W0, W1):
    ea_lin = edge_attr.reshape(EB, 128, 2).transpose(0, 2, 1).reshape(E * 2)
    flat = _edge_encode(ea_lin, W0, W1)
    return flat.reshape(4, EB, 8, 128).transpose(1, 3, 0, 2).reshape(E, OUT_D)


# speedup vs baseline: 1.5015x; 1.1455x over previous
import functools

import jax
import jax.numpy as jnp
from jax import lax
from jax.experimental import pallas as pl
from jax.experimental.pallas import tpu as pltpu
from jax.experimental.pallas import tpu_sc as plsc

E = 3_200_000
EMB = 16
OUT_D = 32
NC = 2
NS = 16
L = 16
NW = NC * NS
EB = E // 128
BPC = 8
CHUNK_E = BPC * 128
IW = BPC * 256
OW = BPC * 1024
NCHUNKS = EB // BPC
CPW = -(-NCHUNKS // NW)
GROUPS = CHUNK_E // L

_mesh = plsc.VectorSubcoreMesh(core_axis_name="c", subcore_axis_name="s")


@functools.partial(
    pl.kernel,
    mesh=_mesh,
    compiler_params=pltpu.CompilerParams(
        needs_layout_passes=False, use_tc_tiling_on_sc=False
    ),
    out_type=jax.ShapeDtypeStruct((E * OUT_D,), jnp.float32),
    scratch_types=[
        pltpu.VMEM((4, EMB), jnp.float32),
        pltpu.VMEM((4, EMB), jnp.float32),
        pltpu.VMEM((L * 65,), jnp.float32),
        pltpu.VMEM((L * 65,), jnp.float32),
        pltpu.VMEM((IW,), jnp.int32),
        pltpu.VMEM((4 * OW,), jnp.float32),
    ],
)
def _edge_encode(
    edge_hbm, w0_hbm, w1_hbm, out_hbm, w0_v, w1_v, w0r_v, w1r_v, idx_v, out_v
):
    wid = lax.axis_index("s") * NC + lax.axis_index("c")
    pltpu.sync_copy(w0_hbm, w0_v)
    pltpu.sync_copy(w1_hbm, w1_v)

    iota = lax.iota(jnp.int32, L)
    lane65 = iota * 65

    for row in range(4):
        v0 = w0_v[row]
        v1 = w1_v[row]
        for l in range(L):
            w0r_v[pl.ds(l * 65 + row * EMB, L)] = v0
            w1r_v[pl.ds(l * 65 + row * EMB, L)] = v1

    def chunk_body(k, carry):
        ci = k * NW + wid

        @pl.when(ci < NCHUNKS)
        def _():
            t0 = ci * BPC
            pltpu.sync_copy(edge_hbm.at[pl.ds(ci * IW, IW)], idx_v)

            @plsc.parallel_loop(0, GROUPS, 1, unroll=2)
            def _grp(g):
                b = g >> 3
                ll0 = (g & 7) << 4
                in_addr = (b * 256 + ll0) + iota
                a0 = plsc.load_gather(idx_v, [in_addr])
                a1 = plsc.load_gather(idx_v, [in_addr + 128])
                base0 = a0 * EMB + lane65
                base1 = a1 * EMB + lane65
                ob = (b * 1024 + ll0) + iota
                for d in range(OUT_D):
                    r, m = d >> 3, d & 7
                    if d < EMB:
                        v = plsc.load_gather(w0r_v, [base0 + d])
                    else:
                        v = plsc.load_gather(w1r_v, [base1 + (d - EMB)])
                    plsc.store_scatter(out_v, [ob + (r * OW + m * 128)], v)

            for r in range(4):
                pltpu.sync_copy(
                    out_v.at[pl.ds(r * OW, OW)],
                    out_hbm.at[pl.ds((r * EB + t0) * 1024, OW)],
                )

        return carry

    lax.fori_loop(0, CPW, chunk_body, 0)


def kernel(edge_attr, W0, W1):
    ea_lin = edge_attr.reshape(EB, 128, 2).transpose(0, 2, 1).reshape(E * 2)
    flat = _edge_encode(ea_lin, W0, W1)
    return flat.reshape(4, EB, 8, 128).transpose(1, 3, 0, 2).reshape(E, OUT_D)


# speedup vs baseline: 2.3888x; 1.5909x over previous
import functools

import jax
import jax.numpy as jnp
from jax import lax
from jax.experimental import pallas as pl
from jax.experimental.pallas import tpu as pltpu
from jax.experimental.pallas import tpu_sc as plsc

E = 3_200_000
EMB = 16
OUT_D = 32
NC = 2
NS = 16
L = 16
NW = NC * NS
EB = E // 128
BPC = 8
CHUNK_E = BPC * 128
IW = BPC * 256
OW = BPC * 1024
NCHUNKS = EB // BPC
CPW = -(-NCHUNKS // NW)
GROUPS = CHUNK_E // L

_mesh = plsc.VectorSubcoreMesh(core_axis_name="c", subcore_axis_name="s")


@functools.partial(
    pl.kernel,
    mesh=_mesh,
    compiler_params=pltpu.CompilerParams(
        needs_layout_passes=False, use_tc_tiling_on_sc=False
    ),
    out_type=jax.ShapeDtypeStruct((E * OUT_D,), jnp.float32),
    scratch_types=[
        pltpu.VMEM((4, EMB), jnp.float32),
        pltpu.VMEM((4, EMB), jnp.float32),
        pltpu.VMEM((L * 65,), jnp.float32),
        pltpu.VMEM((L * 65,), jnp.float32),
        pltpu.VMEM((IW,), jnp.int32),
        pltpu.VMEM((4 * OW,), jnp.float32),
    ],
)
def _edge_encode(
    edge_hbm, w0_hbm, w1_hbm, out_hbm, w0_v, w1_v, w0r_v, w1r_v, idx_v, out_v
):
    wid = lax.axis_index("s") * NC + lax.axis_index("c")
    pltpu.sync_copy(w0_hbm, w0_v)
    pltpu.sync_copy(w1_hbm, w1_v)

    iota = lax.iota(jnp.int32, L)
    lane65 = iota * 65

    for row in range(4):
        v0 = w0_v[row]
        v1 = w1_v[row]
        for l in range(L):
            w0r_v[pl.ds(l * 65 + row * EMB, L)] = v0
            w1r_v[pl.ds(l * 65 + row * EMB, L)] = v1

    def chunk_body(k, carry):
        ci = k * NW + wid

        @pl.when(ci < NCHUNKS)
        def _():
            t0 = ci * BPC
            pltpu.sync_copy(edge_hbm.at[pl.ds(ci * IW, IW)], idx_v)

            @plsc.parallel_loop(0, GROUPS, 1, unroll=1)
            def _grp(g):
                b = g >> 3
                ll0 = (g & 7) << 4
                in_addr = (b * 256 + ll0) + iota
                a0 = plsc.load_gather(idx_v, [in_addr])
                a1 = plsc.load_gather(idx_v, [in_addr + 128])
                base0 = a0 * EMB + lane65
                base1 = a1 * EMB + lane65
                ob = (b * 1024 + ll0) + iota
                for d in range(OUT_D):
                    r, m = d >> 3, d & 7
                    if d < EMB:
                        v = plsc.load_gather(w0r_v, [base0 + d])
                    else:
                        v = plsc.load_gather(w1r_v, [base1 + (d - EMB)])
                    plsc.store_scatter(out_v, [ob + (r * OW + m * 128)], v)

            for r in range(4):
                pltpu.sync_copy(
                    out_v.at[pl.ds(r * OW, OW)],
                    out_hbm.at[pl.ds((r * EB + t0) * 1024, OW)],
                )

        return carry

    lax.fori_loop(0, CPW, chunk_body, 0)


def kernel(edge_attr, W0, W1):
    ea_lin = edge_attr.reshape(EB, 128, 2).transpose(0, 2, 1).reshape(E * 2)
    flat = _edge_encode(ea_lin, W0, W1)
    return flat.reshape(4, EB, 8, 128).transpose(1, 3, 0, 2).reshape(E, OUT_D)


# speedup vs baseline: 4.0931x; 1.7134x over previous
import functools

import jax
import jax.numpy as jnp
from jax import lax
from jax.experimental import pallas as pl
from jax.experimental.pallas import tpu as pltpu
from jax.experimental.pallas import tpu_sc as plsc

E = 3_200_000
EMB = 16
OUT_D = 32
NC = 2
NS = 16
L = 16
NW = NC * NS
EB = E // 128
BPC = 8
CHUNK_E = BPC * 128
IW = BPC * 256
OW = BPC * 1024
NCHUNKS = EB // BPC
CPW = -(-NCHUNKS // NW)
GROUPS = CHUNK_E // L

_mesh = plsc.VectorSubcoreMesh(core_axis_name="c", subcore_axis_name="s")


@functools.partial(
    pl.kernel,
    mesh=_mesh,
    compiler_params=pltpu.CompilerParams(
        needs_layout_passes=False, use_tc_tiling_on_sc=False
    ),
    out_type=jax.ShapeDtypeStruct((E * OUT_D,), jnp.float32),
    scratch_types=[
        pltpu.VMEM((4, EMB), jnp.float32),
        pltpu.VMEM((4, EMB), jnp.float32),
        pltpu.VMEM((L * 65,), jnp.float32),
        pltpu.VMEM((L * 65,), jnp.float32),
        pltpu.VMEM((IW,), jnp.int32),
        pltpu.VMEM((IW,), jnp.int32),
        pltpu.VMEM((4 * OW,), jnp.float32),
        pltpu.VMEM((4 * OW,), jnp.float32),
        pltpu.SemaphoreType.DMA,
        pltpu.SemaphoreType.DMA,
        pltpu.SemaphoreType.DMA,
        pltpu.SemaphoreType.DMA,
    ],
)
def _edge_encode(
    edge_hbm, w0_hbm, w1_hbm, out_hbm, w0_v, w1_v, w0r_v, w1r_v,
    idx0_v, idx1_v, out0_v, out1_v, si0, si1, so0, so1,
):
    wid = lax.axis_index("s") * NC + lax.axis_index("c")
    pltpu.sync_copy(w0_hbm, w0_v)
    pltpu.sync_copy(w1_hbm, w1_v)

    iota = lax.iota(jnp.int32, L)
    lane65 = iota * 65

    for row in range(4):
        v0 = w0_v[row]
        v1 = w1_v[row]
        for l in range(L):
            w0r_v[pl.ds(l * 65 + row * EMB, L)] = v0
            w1r_v[pl.ds(l * 65 + row * EMB, L)] = v1

    n_w = NCHUNKS // NW + jnp.where(wid < NCHUNKS % NW, 1, 0)
    idx_bufs = (idx0_v, idx1_v)
    out_bufs = (out0_v, out1_v)
    in_sems = (si0, si1)
    out_sems = (so0, so1)

    def in_src(k):
        return edge_hbm.at[pl.ds((k * NW + wid) * IW, IW)]

    def compute(idx_v, out_v):
        @plsc.parallel_loop(0, GROUPS, 1, unroll=1)
        def _grp(g):
            b = g >> 3
            ll0 = (g & 7) << 4
            in_addr = (b * 256 + ll0) + iota
            a0 = plsc.load_gather(idx_v, [in_addr])
            a1 = plsc.load_gather(idx_v, [in_addr + 128])
            base0 = a0 * EMB + lane65
            base1 = a1 * EMB + lane65
            ob = (b * 1024 + ll0) + iota
            for d in range(OUT_D):
                r, m = d >> 3, d & 7
                if d < EMB:
                    v = plsc.load_gather(w0r_v, [base0 + d])
                else:
                    v = plsc.load_gather(w1r_v, [base1 + (d - EMB)])
                plsc.store_scatter(out_v, [ob + (r * OW + m * 128)], v)

    def out_dma(p, k, issue):
        t0 = (k * NW + wid) * BPC
        for r in range(4):
            desc = pltpu.make_async_copy(
                out_bufs[p].at[pl.ds(r * OW, OW)],
                out_hbm.at[pl.ds((r * EB + t0) * 1024, OW)],
                out_sems[p],
            )
            if issue:
                desc.start()
            else:
                desc.wait()

    pltpu.async_copy(in_src(0), idx0_v, si0)
    pltpu.async_copy(in_src(1), idx1_v, si1)

    def body(j, carry):
        for p in range(2):
            k = j * 2 + p

            @pl.when(k < n_w)
            def _():
                pltpu.make_async_copy(in_src(k), idx_bufs[p], in_sems[p]).wait()

                @pl.when(k >= 2)
                def _():
                    out_dma(p, k - 2, issue=False)

                compute(idx_bufs[p], out_bufs[p])
                out_dma(p, k, issue=True)

                @pl.when(k + 2 < n_w)
                def _():
                    pltpu.async_copy(in_src(k + 2), idx_bufs[p], in_sems[p])

        return carry

    lax.fori_loop(0, (CPW + 1) // 2, body, 0)

    for p in range(2):
        out_dma(p, ((n_w - 1 - p) // 2) * 2 + p, issue=False)


def kernel(edge_attr, W0, W1):
    ea_lin = edge_attr.reshape(EB, 128, 2).transpose(0, 2, 1).reshape(E * 2)
    flat = _edge_encode(ea_lin, W0, W1)
    return flat.reshape(4, EB, 8, 128).transpose(1, 3, 0, 2).reshape(E, OUT_D)
